# baseline (speedup 1.0000x reference)
import functools

import jax
import jax.numpy as jnp
from jax import lax
from jax.experimental import pallas as pl
from jax.experimental.pallas import tpu as pltpu
from jax.experimental.pallas import tpu_sc as plsc

N_NODES = 10000
N_EDGES = 320000
D = 128

NC = 2
NS = 16
NW = NC * NS
NP = 10240
CHUNK = 128
CPW = 80
AC = 128
A0 = 48
A1 = 112
AGC = 8
EP = NW * CPW * CHUNK
ROWS_PER_TILE = NP // NS

_F32 = jnp.float32


@functools.cache
def _mesh():
    return plsc.VectorSubcoreMesh(
        core_axis_name="c", subcore_axis_name="s", num_cores=NC, num_subcores=NS
    )


def _deg_body(dst_hbm, out_hbm, idx_v, cnt_v):
    c = lax.axis_index("c")
    s = lax.axis_index("s")
    w = c * NS + s
    pltpu.sync_copy(dst_hbm.at[pl.ds(w * CPW, CPW)], idx_v)

    def zero_body(i, carry):
        cnt_v[pl.ds(i * 16, 16)] = jnp.zeros((16,), _F32)
        return carry

    lax.fori_loop(0, NP // 16, zero_body, 0)

    ones = jnp.ones((16,), _F32)

    def scat_body(j, carry):
        for k in range(CHUNK // 16):
            idx16 = idx_v[j, pl.ds(k * 16, 16)]
            plsc.addupdate_scatter(cnt_v, [idx16], ones)
        return carry

    lax.fori_loop(0, CPW, scat_body, 0)
    pltpu.sync_copy(cnt_v, out_hbm.at[w])


@functools.cache
def _deg_call():
    return pl.kernel(
        _deg_body,
        out_type=jax.ShapeDtypeStruct((NW, NP), _F32),
        mesh=_mesh(),
        scratch_types=[
            pltpu.VMEM((CPW, CHUNK), jnp.int32),
            pltpu.VMEM((NP,), _F32),
        ],
        compiler_params=pltpu.CompilerParams(needs_layout_passes=False),
    )


def _agg_body(
    g_hbm, src_hbm, dst_hbm, out_hbm, sidx_v, didx_v, rows_a, rows_b, agg_sh, sem_a, sem_b
):
    c = lax.axis_index("c")
    s = lax.axis_index("s")
    nch = jnp.where(c == 0, A0, A1)
    base_ch = c * NS * A0 + s * nch

    def zrow(i, carry):
        for k in range(D // 16):
            rows_a[i, pl.ds(k * 16, 16)] = jnp.zeros((16,), _F32)
        return carry

    lax.fori_loop(0, AC, zrow, 0)
    base = s * ROWS_PER_TILE
    for k in range(ROWS_PER_TILE // AC):
        pltpu.sync_copy(rows_a, agg_sh.at[pl.ds(base + k * AC, AC)])
    plsc.subcore_barrier()

    def group_body(gi, carry):
        gb = base_ch + gi * AGC
        pltpu.sync_copy(src_hbm.at[pl.ds(gb, AGC)], sidx_v)
        pltpu.sync_copy(dst_hbm.at[pl.ds(gb, AGC)], didx_v)
        pltpu.async_copy(g_hbm.at[sidx_v.at[0]], rows_a, sem_a)

        def chunk_pair(j2, carry2):
            j = j2 * 2
            pltpu.make_async_copy(g_hbm.at[sidx_v.at[j]], rows_a, sem_a).wait()
            pltpu.async_copy(g_hbm.at[sidx_v.at[j + 1]], rows_b, sem_b)
            pltpu.sync_copy(rows_a, agg_sh.at[didx_v.at[j]], add=True)
            pltpu.make_async_copy(g_hbm.at[sidx_v.at[j + 1]], rows_b, sem_b).wait()

            @pl.when(j + 2 < AGC)
            def _():
                pltpu.async_copy(g_hbm.at[sidx_v.at[j + 2]], rows_a, sem_a)

            pltpu.sync_copy(rows_b, agg_sh.at[didx_v.at[j + 1]], add=True)
            return carry2

        lax.fori_loop(0, AGC // 2, chunk_pair, 0)
        return carry

    lax.fori_loop(0, nch // AGC, group_body, 0)
    plsc.subcore_barrier()

    for k in range(ROWS_PER_TILE // AC):
        pltpu.sync_copy(agg_sh.at[pl.ds(base + k * AC, AC)], rows_a)
        pltpu.sync_copy(rows_a, out_hbm.at[c, pl.ds(base + k * AC, AC)])


@functools.cache
def _agg_call():
    return pl.kernel(
        _agg_body,
        out_type=jax.ShapeDtypeStruct((NC, NP, D), _F32),
        mesh=_mesh(),
        scratch_types=[
            pltpu.VMEM((AGC, AC), jnp.int32),
            pltpu.VMEM((AGC, AC), jnp.int32),
            pltpu.VMEM((AC, D), _F32),
            pltpu.VMEM((AC, D), _F32),
            pltpu.VMEM_SHARED((NP, D), _F32),
            pltpu.SemaphoreType.DMA,
            pltpu.SemaphoreType.DMA,
        ],
    )


_R = 1024


def _tc1_body(x_ref, pw_ref, pb_ref, w1_ref, degp_ref, g_ref, dinv_ref):
    h = jnp.dot(x_ref[...], pw_ref[...], preferred_element_type=_F32)
    h = h + pb_ref[...][None, :]
    deg = 1.0 + jnp.sum(degp_ref[...], axis=0)
    dinv = lax.rsqrt(deg)
    g_ref[...] = jnp.dot(h, w1_ref[...], preferred_element_type=_F32) * dinv[:, None]
    dinv_ref[...] = dinv[:, None]


_tc1_call = pl.pallas_call(
    _tc1_body,
    grid=(NP // _R,),
    in_specs=[
        pl.BlockSpec((_R, D), lambda i: (i, 0)),
        pl.BlockSpec((D, D), lambda i: (0, 0)),
        pl.BlockSpec((D,), lambda i: (0,)),
        pl.BlockSpec((D, D), lambda i: (0, 0)),
        pl.BlockSpec((NW, _R), lambda i: (0, i)),
    ],
    out_specs=[
        pl.BlockSpec((_R, D), lambda i: (i, 0)),
        pl.BlockSpec((_R, 1), lambda i: (i, 0)),
    ],
    out_shape=[
        jax.ShapeDtypeStruct((NP, D), _F32),
        jax.ShapeDtypeStruct((NP, 1), _F32),
    ],
)


def _tc_mid_body(p_ref, g_ref, dinv_ref, b_ref, w_ref, gout_ref):
    p = p_ref[...]
    agg = (p[0] + p[1] + g_ref[...]) * dinv_ref[...]
    h = jnp.maximum(agg + b_ref[...][None, :], 0.0)
    gout_ref[...] = jnp.dot(h, w_ref[...], preferred_element_type=_F32) * dinv_ref[...]


_tc_mid_call = pl.pallas_call(
    _tc_mid_body,
    grid=(NP // _R,),
    in_specs=[
        pl.BlockSpec((NC, _R, D), lambda i: (0, i, 0)),
        pl.BlockSpec((_R, D), lambda i: (i, 0)),
        pl.BlockSpec((_R, 1), lambda i: (i, 0)),
        pl.BlockSpec((D,), lambda i: (0,)),
        pl.BlockSpec((D, D), lambda i: (0, 0)),
    ],
    out_specs=pl.BlockSpec((_R, D), lambda i: (i, 0)),
    out_shape=jax.ShapeDtypeStruct((NP, D), _F32),
)


def _tc_out_body(p_ref, g_ref, dinv_ref, b_ref, w_ref, ob_ref, out_ref):
    p = p_ref[...]
    agg = (p[0] + p[1] + g_ref[...]) * dinv_ref[...]
    h = jnp.maximum(agg + b_ref[...][None, :], 0.0)
    out_ref[...] = (
        jnp.dot(h, w_ref[...], preferred_element_type=_F32) + ob_ref[...][None, :]
    )


_tc_out_call = pl.pallas_call(
    _tc_out_body,
    grid=(NP // _R,),
    in_specs=[
        pl.BlockSpec((NC, _R, D), lambda i: (0, i, 0)),
        pl.BlockSpec((_R, D), lambda i: (i, 0)),
        pl.BlockSpec((_R, 1), lambda i: (i, 0)),
        pl.BlockSpec((D,), lambda i: (0,)),
        pl.BlockSpec((D, D), lambda i: (0, 0)),
        pl.BlockSpec((D,), lambda i: (0,)),
    ],
    out_specs=pl.BlockSpec((_R, D), lambda i: (i, 0)),
    out_shape=jax.ShapeDtypeStruct((NP, D), _F32),
)


def kernel(x, edge_index, proj_W, proj_b, W1, b1, W2, b2, out_W, out_b):
    src = edge_index[0].astype(jnp.int32)
    dst = edge_index[1].astype(jnp.int32)
    pad_e = EP - N_EDGES
    pad_idx = jnp.full((pad_e,), N_NODES, jnp.int32)
    src_flat = jnp.concatenate([src, pad_idx])
    dst_flat = jnp.concatenate([dst, pad_idx])
    dstp_deg = dst_flat.reshape(NW * CPW, CHUNK)
    srcp = src_flat.reshape(NW * CPW, AC)
    dstp = dst_flat.reshape(NW * CPW, AC)
    xp = jnp.pad(x, ((0, NP - N_NODES), (0, 0)))

    degp = _deg_call()(dstp_deg)
    g1, dinv = _tc1_call(xp, proj_W, proj_b, W1, degp)
    parts1 = _agg_call()(g1, srcp, dstp)
    g2 = _tc_mid_call(parts1, g1, dinv, b1, W2)
    parts2 = _agg_call()(g2, srcp, dstp)
    out = _tc_out_call(parts2, g2, dinv, b2, out_W, out_b)
    return out[:N_NODES]


# speedup vs baseline: 1.1647x; 1.1647x over previous
import functools

import jax
import jax.numpy as jnp
from jax import lax
from jax.experimental import pallas as pl
from jax.experimental.pallas import tpu as pltpu
from jax.experimental.pallas import tpu_sc as plsc

N_NODES = 10000
N_EDGES = 320000
D = 128

NC = 2
NS = 16
NW = NC * NS
NP = 10240
CHUNK = 128
CPW = 80
AC = 128
A0 = 112
A1 = 48
AGC = 8
EP = NW * CPW * CHUNK
ROWS_PER_TILE = NP // NS

_F32 = jnp.float32


@functools.cache
def _mesh():
    return plsc.VectorSubcoreMesh(
        core_axis_name="c", subcore_axis_name="s", num_cores=NC, num_subcores=NS
    )


def _deg_body(dst_hbm, out_hbm, idx_v, cnt_v):
    c = lax.axis_index("c")
    s = lax.axis_index("s")
    w = c * NS + s
    pltpu.sync_copy(dst_hbm.at[pl.ds(w * CPW, CPW)], idx_v)

    def zero_body(i, carry):
        cnt_v[pl.ds(i * 16, 16)] = jnp.zeros((16,), _F32)
        return carry

    lax.fori_loop(0, NP // 16, zero_body, 0)

    ones = jnp.ones((16,), _F32)

    def scat_body(j, carry):
        for k in range(CHUNK // 16):
            idx16 = idx_v[j, pl.ds(k * 16, 16)]
            plsc.addupdate_scatter(cnt_v, [idx16], ones)
        return carry

    lax.fori_loop(0, CPW, scat_body, 0)
    pltpu.sync_copy(cnt_v, out_hbm.at[w])


@functools.cache
def _deg_call():
    return pl.kernel(
        _deg_body,
        out_type=jax.ShapeDtypeStruct((NW, NP), _F32),
        mesh=_mesh(),
        scratch_types=[
            pltpu.VMEM((CPW, CHUNK), jnp.int32),
            pltpu.VMEM((NP,), _F32),
        ],
        compiler_params=pltpu.CompilerParams(needs_layout_passes=False),
    )


def _agg_body(
    g_hbm, src_hbm, dst_hbm, out_hbm, sidx_v, didx_v, rows_a, rows_b, agg_sh, sem_a, sem_b
):
    c = lax.axis_index("c")
    s = lax.axis_index("s")
    nch = jnp.where(c == 0, A0, A1)
    base_ch = c * NS * A0 + s * nch

    def zrow(i, carry):
        for k in range(D // 16):
            rows_a[i, pl.ds(k * 16, 16)] = jnp.zeros((16,), _F32)
        return carry

    lax.fori_loop(0, AC, zrow, 0)
    base = s * ROWS_PER_TILE
    for k in range(ROWS_PER_TILE // AC):
        pltpu.sync_copy(rows_a, agg_sh.at[pl.ds(base + k * AC, AC)])
    plsc.subcore_barrier()

    def group_body(gi, carry):
        gb = base_ch + gi * AGC
        pltpu.sync_copy(src_hbm.at[pl.ds(gb, AGC)], sidx_v)
        pltpu.sync_copy(dst_hbm.at[pl.ds(gb, AGC)], didx_v)
        pltpu.async_copy(g_hbm.at[sidx_v.at[0]], rows_a, sem_a)

        def chunk_pair(j2, carry2):
            j = j2 * 2
            pltpu.make_async_copy(g_hbm.at[sidx_v.at[j]], rows_a, sem_a).wait()
            pltpu.async_copy(g_hbm.at[sidx_v.at[j + 1]], rows_b, sem_b)
            pltpu.sync_copy(rows_a, agg_sh.at[didx_v.at[j]], add=True)
            pltpu.make_async_copy(g_hbm.at[sidx_v.at[j + 1]], rows_b, sem_b).wait()

            @pl.when(j + 2 < AGC)
            def _():
                pltpu.async_copy(g_hbm.at[sidx_v.at[j + 2]], rows_a, sem_a)

            pltpu.sync_copy(rows_b, agg_sh.at[didx_v.at[j + 1]], add=True)
            return carry2

        lax.fori_loop(0, AGC // 2, chunk_pair, 0)
        return carry

    lax.fori_loop(0, nch // AGC, group_body, 0)
    plsc.subcore_barrier()

    for k in range(ROWS_PER_TILE // AC):
        pltpu.sync_copy(agg_sh.at[pl.ds(base + k * AC, AC)], rows_a)
        pltpu.sync_copy(rows_a, out_hbm.at[c, pl.ds(base + k * AC, AC)])


@functools.cache
def _agg_call():
    return pl.kernel(
        _agg_body,
        out_type=jax.ShapeDtypeStruct((NC, NP, D), _F32),
        mesh=_mesh(),
        scratch_types=[
            pltpu.VMEM((AGC, AC), jnp.int32),
            pltpu.VMEM((AGC, AC), jnp.int32),
            pltpu.VMEM((AC, D), _F32),
            pltpu.VMEM((AC, D), _F32),
            pltpu.VMEM_SHARED((NP, D), _F32),
            pltpu.SemaphoreType.DMA,
            pltpu.SemaphoreType.DMA,
        ],
    )


_R = 1024


def _tc1_body(x_ref, pw_ref, pb_ref, w1_ref, degp_ref, g_ref, dinv_ref):
    h = jnp.dot(x_ref[...], pw_ref[...], preferred_element_type=_F32)
    h = h + pb_ref[...][None, :]
    deg = 1.0 + jnp.sum(degp_ref[...], axis=0)
    dinv = lax.rsqrt(deg)
    g_ref[...] = jnp.dot(h, w1_ref[...], preferred_element_type=_F32) * dinv[:, None]
    dinv_ref[...] = dinv[:, None]


_tc1_call = pl.pallas_call(
    _tc1_body,
    grid=(NP // _R,),
    in_specs=[
        pl.BlockSpec((_R, D), lambda i: (i, 0)),
        pl.BlockSpec((D, D), lambda i: (0, 0)),
        pl.BlockSpec((D,), lambda i: (0,)),
        pl.BlockSpec((D, D), lambda i: (0, 0)),
        pl.BlockSpec((NW, _R), lambda i: (0, i)),
    ],
    out_specs=[
        pl.BlockSpec((_R, D), lambda i: (i, 0)),
        pl.BlockSpec((_R, 1), lambda i: (i, 0)),
    ],
    out_shape=[
        jax.ShapeDtypeStruct((NP, D), _F32),
        jax.ShapeDtypeStruct((NP, 1), _F32),
    ],
)


def _tc_mid_body(p_ref, g_ref, dinv_ref, b_ref, w_ref, gout_ref):
    p = p_ref[...]
    agg = (p[0] + p[1] + g_ref[...]) * dinv_ref[...]
    h = jnp.maximum(agg + b_ref[...][None, :], 0.0)
    gout_ref[...] = jnp.dot(h, w_ref[...], preferred_element_type=_F32) * dinv_ref[...]


_tc_mid_call = pl.pallas_call(
    _tc_mid_body,
    grid=(NP // _R,),
    in_specs=[
        pl.BlockSpec((NC, _R, D), lambda i: (0, i, 0)),
        pl.BlockSpec((_R, D), lambda i: (i, 0)),
        pl.BlockSpec((_R, 1), lambda i: (i, 0)),
        pl.BlockSpec((D,), lambda i: (0,)),
        pl.BlockSpec((D, D), lambda i: (0, 0)),
    ],
    out_specs=pl.BlockSpec((_R, D), lambda i: (i, 0)),
    out_shape=jax.ShapeDtypeStruct((NP, D), _F32),
)


def _tc_out_body(p_ref, g_ref, dinv_ref, b_ref, w_ref, ob_ref, out_ref):
    p = p_ref[...]
    agg = (p[0] + p[1] + g_ref[...]) * dinv_ref[...]
    h = jnp.maximum(agg + b_ref[...][None, :], 0.0)
    out_ref[...] = (
        jnp.dot(h, w_ref[...], preferred_element_type=_F32) + ob_ref[...][None, :]
    )


_tc_out_call = pl.pallas_call(
    _tc_out_body,
    grid=(NP // _R,),
    in_specs=[
        pl.BlockSpec((NC, _R, D), lambda i: (0, i, 0)),
        pl.BlockSpec((_R, D), lambda i: (i, 0)),
        pl.BlockSpec((_R, 1), lambda i: (i, 0)),
        pl.BlockSpec((D,), lambda i: (0,)),
        pl.BlockSpec((D, D), lambda i: (0, 0)),
        pl.BlockSpec((D,), lambda i: (0,)),
    ],
    out_specs=pl.BlockSpec((_R, D), lambda i: (i, 0)),
    out_shape=jax.ShapeDtypeStruct((NP, D), _F32),
)


def kernel(x, edge_index, proj_W, proj_b, W1, b1, W2, b2, out_W, out_b):
    src = edge_index[0].astype(jnp.int32)
    dst = edge_index[1].astype(jnp.int32)
    pad_e = EP - N_EDGES
    pad_idx = jnp.full((pad_e,), N_NODES, jnp.int32)
    src_flat = jnp.concatenate([src, pad_idx])
    dst_flat = jnp.concatenate([dst, pad_idx])
    dstp_deg = dst_flat.reshape(NW * CPW, CHUNK)
    srcp = src_flat.reshape(NW * CPW, AC)
    dstp = dst_flat.reshape(NW * CPW, AC)
    xp = jnp.pad(x, ((0, NP - N_NODES), (0, 0)))

    degp = _deg_call()(dstp_deg)
    g1, dinv = _tc1_call(xp, proj_W, proj_b, W1, degp)
    parts1 = _agg_call()(g1, srcp, dstp)
    g2 = _tc_mid_call(parts1, g1, dinv, b1, W2)
    parts2 = _agg_call()(g2, srcp, dstp)
    out = _tc_out_call(parts2, g2, dinv, b2, out_W, out_b)
    return out[:N_NODES]


# speedup vs baseline: 1.1893x; 1.0211x over previous
import functools

import jax
import jax.numpy as jnp
from jax import lax
from jax.experimental import pallas as pl
from jax.experimental.pallas import tpu as pltpu
from jax.experimental.pallas import tpu_sc as plsc

N_NODES = 10000
N_EDGES = 320000
D = 128

NC = 2
NS = 16
NW = NC * NS
NP = 10240
CHUNK = 128
CPW = 80
AC = 128
A0 = 120
A1 = 40
AGC = 8
EP = NW * CPW * CHUNK
ROWS_PER_TILE = NP // NS

_F32 = jnp.float32


@functools.cache
def _mesh():
    return plsc.VectorSubcoreMesh(
        core_axis_name="c", subcore_axis_name="s", num_cores=NC, num_subcores=NS
    )


def _deg_body(dst_hbm, out_hbm, idx_v, cnt_v):
    c = lax.axis_index("c")
    s = lax.axis_index("s")
    w = c * NS + s
    pltpu.sync_copy(dst_hbm.at[pl.ds(w * CPW, CPW)], idx_v)

    def zero_body(i, carry):
        cnt_v[pl.ds(i * 16, 16)] = jnp.zeros((16,), _F32)
        return carry

    lax.fori_loop(0, NP // 16, zero_body, 0)

    ones = jnp.ones((16,), _F32)

    def scat_body(j, carry):
        for k in range(CHUNK // 16):
            idx16 = idx_v[j, pl.ds(k * 16, 16)]
            plsc.addupdate_scatter(cnt_v, [idx16], ones)
        return carry

    lax.fori_loop(0, CPW, scat_body, 0)
    pltpu.sync_copy(cnt_v, out_hbm.at[w])


@functools.cache
def _deg_call():
    return pl.kernel(
        _deg_body,
        out_type=jax.ShapeDtypeStruct((NW, NP), _F32),
        mesh=_mesh(),
        scratch_types=[
            pltpu.VMEM((CPW, CHUNK), jnp.int32),
            pltpu.VMEM((NP,), _F32),
        ],
        compiler_params=pltpu.CompilerParams(needs_layout_passes=False),
    )


def _agg_body(
    g_hbm, src_hbm, dst_hbm, out_hbm, sidx_v, didx_v, rows_a, rows_b, agg_sh, sem_a, sem_b
):
    c = lax.axis_index("c")
    s = lax.axis_index("s")
    nch = jnp.where(c == 0, A0, A1)
    base_ch = c * NS * A0 + s * nch

    def zrow(i, carry):
        for k in range(D // 16):
            rows_a[i, pl.ds(k * 16, 16)] = jnp.zeros((16,), _F32)
        return carry

    lax.fori_loop(0, AC, zrow, 0)
    base = s * ROWS_PER_TILE
    for k in range(ROWS_PER_TILE // AC):
        pltpu.sync_copy(rows_a, agg_sh.at[pl.ds(base + k * AC, AC)])
    plsc.subcore_barrier()

    def group_body(gi, carry):
        gb = base_ch + gi * AGC
        pltpu.sync_copy(src_hbm.at[pl.ds(gb, AGC)], sidx_v)
        pltpu.sync_copy(dst_hbm.at[pl.ds(gb, AGC)], didx_v)
        pltpu.async_copy(g_hbm.at[sidx_v.at[0]], rows_a, sem_a)

        def chunk_pair(j2, carry2):
            j = j2 * 2
            pltpu.make_async_copy(g_hbm.at[sidx_v.at[j]], rows_a, sem_a).wait()
            pltpu.async_copy(g_hbm.at[sidx_v.at[j + 1]], rows_b, sem_b)
            pltpu.sync_copy(rows_a, agg_sh.at[didx_v.at[j]], add=True)
            pltpu.make_async_copy(g_hbm.at[sidx_v.at[j + 1]], rows_b, sem_b).wait()

            @pl.when(j + 2 < AGC)
            def _():
                pltpu.async_copy(g_hbm.at[sidx_v.at[j + 2]], rows_a, sem_a)

            pltpu.sync_copy(rows_b, agg_sh.at[didx_v.at[j + 1]], add=True)
            return carry2

        lax.fori_loop(0, AGC // 2, chunk_pair, 0)
        return carry

    lax.fori_loop(0, nch // AGC, group_body, 0)
    plsc.subcore_barrier()

    for k in range(ROWS_PER_TILE // AC):
        pltpu.sync_copy(agg_sh.at[pl.ds(base + k * AC, AC)], rows_a)
        pltpu.sync_copy(rows_a, out_hbm.at[c, pl.ds(base + k * AC, AC)])


@functools.cache
def _agg_call():
    return pl.kernel(
        _agg_body,
        out_type=jax.ShapeDtypeStruct((NC, NP, D), _F32),
        mesh=_mesh(),
        scratch_types=[
            pltpu.VMEM((AGC, AC), jnp.int32),
            pltpu.VMEM((AGC, AC), jnp.int32),
            pltpu.VMEM((AC, D), _F32),
            pltpu.VMEM((AC, D), _F32),
            pltpu.VMEM_SHARED((NP, D), _F32),
            pltpu.SemaphoreType.DMA,
            pltpu.SemaphoreType.DMA,
        ],
    )


_R = 1024


def _tc1_body(x_ref, pw_ref, pb_ref, w1_ref, degp_ref, g_ref, dinv_ref):
    h = jnp.dot(x_ref[...], pw_ref[...], preferred_element_type=_F32)
    h = h + pb_ref[...][None, :]
    deg = 1.0 + jnp.sum(degp_ref[...], axis=0)
    dinv = lax.rsqrt(deg)
    g_ref[...] = jnp.dot(h, w1_ref[...], preferred_element_type=_F32) * dinv[:, None]
    dinv_ref[...] = dinv[:, None]


_tc1_call = pl.pallas_call(
    _tc1_body,
    grid=(NP // _R,),
    in_specs=[
        pl.BlockSpec((_R, D), lambda i: (i, 0)),
        pl.BlockSpec((D, D), lambda i: (0, 0)),
        pl.BlockSpec((D,), lambda i: (0,)),
        pl.BlockSpec((D, D), lambda i: (0, 0)),
        pl.BlockSpec((NW, _R), lambda i: (0, i)),
    ],
    out_specs=[
        pl.BlockSpec((_R, D), lambda i: (i, 0)),
        pl.BlockSpec((_R, 1), lambda i: (i, 0)),
    ],
    out_shape=[
        jax.ShapeDtypeStruct((NP, D), _F32),
        jax.ShapeDtypeStruct((NP, 1), _F32),
    ],
)


def _tc_mid_body(p_ref, g_ref, dinv_ref, b_ref, w_ref, gout_ref):
    p = p_ref[...]
    agg = (p[0] + p[1] + g_ref[...]) * dinv_ref[...]
    h = jnp.maximum(agg + b_ref[...][None, :], 0.0)
    gout_ref[...] = jnp.dot(h, w_ref[...], preferred_element_type=_F32) * dinv_ref[...]


_tc_mid_call = pl.pallas_call(
    _tc_mid_body,
    grid=(NP // _R,),
    in_specs=[
        pl.BlockSpec((NC, _R, D), lambda i: (0, i, 0)),
        pl.BlockSpec((_R, D), lambda i: (i, 0)),
        pl.BlockSpec((_R, 1), lambda i: (i, 0)),
        pl.BlockSpec((D,), lambda i: (0,)),
        pl.BlockSpec((D, D), lambda i: (0, 0)),
    ],
    out_specs=pl.BlockSpec((_R, D), lambda i: (i, 0)),
    out_shape=jax.ShapeDtypeStruct((NP, D), _F32),
)


def _tc_out_body(p_ref, g_ref, dinv_ref, b_ref, w_ref, ob_ref, out_ref):
    p = p_ref[...]
    agg = (p[0] + p[1] + g_ref[...]) * dinv_ref[...]
    h = jnp.maximum(agg + b_ref[...][None, :], 0.0)
    out_ref[...] = (
        jnp.dot(h, w_ref[...], preferred_element_type=_F32) + ob_ref[...][None, :]
    )


_tc_out_call = pl.pallas_call(
    _tc_out_body,
    grid=(NP // _R,),
    in_specs=[
        pl.BlockSpec((NC, _R, D), lambda i: (0, i, 0)),
        pl.BlockSpec((_R, D), lambda i: (i, 0)),
        pl.BlockSpec((_R, 1), lambda i: (i, 0)),
        pl.BlockSpec((D,), lambda i: (0,)),
        pl.BlockSpec((D, D), lambda i: (0, 0)),
        pl.BlockSpec((D,), lambda i: (0,)),
    ],
    out_specs=pl.BlockSpec((_R, D), lambda i: (i, 0)),
    out_shape=jax.ShapeDtypeStruct((NP, D), _F32),
)


def kernel(x, edge_index, proj_W, proj_b, W1, b1, W2, b2, out_W, out_b):
    src = edge_index[0].astype(jnp.int32)
    dst = edge_index[1].astype(jnp.int32)
    pad_e = EP - N_EDGES
    pad_idx = jnp.full((pad_e,), N_NODES, jnp.int32)
    src_flat = jnp.concatenate([src, pad_idx])
    dst_flat = jnp.concatenate([dst, pad_idx])
    dstp_deg = dst_flat.reshape(NW * CPW, CHUNK)
    srcp = src_flat.reshape(NW * CPW, AC)
    dstp = dst_flat.reshape(NW * CPW, AC)
    xp = jnp.pad(x, ((0, NP - N_NODES), (0, 0)))

    degp = _deg_call()(dstp_deg)
    g1, dinv = _tc1_call(xp, proj_W, proj_b, W1, degp)
    parts1 = _agg_call()(g1, srcp, dstp)
    g2 = _tc_mid_call(parts1, g1, dinv, b1, W2)
    parts2 = _agg_call()(g2, srcp, dstp)
    out = _tc_out_call(parts2, g2, dinv, b2, out_W, out_b)
    return out[:N_NODES]


# speedup vs baseline: 1.5123x; 1.2716x over previous
import functools

import jax
import jax.numpy as jnp
from jax import lax
from jax.experimental import pallas as pl
from jax.experimental.pallas import tpu as pltpu
from jax.experimental.pallas import tpu_sc as plsc

N_NODES = 10000
N_EDGES = 320000
D = 128

NC = 2
NS = 16
NW = NC * NS
NP = 10240
CHUNK = 128
CPW = 80
AC = 128
A0 = 120
A1 = 40
AGC = 16
EP = NW * CPW * CHUNK
ROWS_PER_TILE = NP // NS

_F32 = jnp.float32


@functools.cache
def _mesh():
    return plsc.VectorSubcoreMesh(
        core_axis_name="c", subcore_axis_name="s", num_cores=NC, num_subcores=NS
    )


def _deg_body(dst_hbm, out_hbm, idx_v, cnt_v):
    c = lax.axis_index("c")
    s = lax.axis_index("s")
    w = c * NS + s
    pltpu.sync_copy(dst_hbm.at[pl.ds(w * CPW, CPW)], idx_v)

    def zero_body(i, carry):
        cnt_v[pl.ds(i * 16, 16)] = jnp.zeros((16,), _F32)
        return carry

    lax.fori_loop(0, NP // 16, zero_body, 0)

    ones = jnp.ones((16,), _F32)

    def scat_body(j, carry):
        for k in range(CHUNK // 16):
            idx16 = idx_v[j, pl.ds(k * 16, 16)]
            plsc.addupdate_scatter(cnt_v, [idx16], ones)
        return carry

    lax.fori_loop(0, CPW, scat_body, 0)
    pltpu.sync_copy(cnt_v, out_hbm.at[w])


@functools.cache
def _deg_call():
    return pl.kernel(
        _deg_body,
        out_type=jax.ShapeDtypeStruct((NW, NP), _F32),
        mesh=_mesh(),
        scratch_types=[
            pltpu.VMEM((CPW, CHUNK), jnp.int32),
            pltpu.VMEM((NP,), _F32),
        ],
        compiler_params=pltpu.CompilerParams(needs_layout_passes=False),
    )


def _agg_body(
    g_hbm, src_hbm, dst_hbm, out_hbm, sidx_v, didx_v, rows_a, rows_b, agg_sh, sem_a, sem_b
):
    c = lax.axis_index("c")
    s = lax.axis_index("s")
    nch = jnp.where(c == 0, A0, A1)
    base_ch = c * NS * A0 + s * nch

    def zrow(i, carry):
        for k in range(D // 16):
            rows_a[i, pl.ds(k * 16, 16)] = jnp.zeros((16,), _F32)
        return carry

    lax.fori_loop(0, AC, zrow, 0)
    base = s * ROWS_PER_TILE
    for k in range(ROWS_PER_TILE // AC):
        pltpu.sync_copy(rows_a, agg_sh.at[pl.ds(base + k * AC, AC)])
    plsc.subcore_barrier()

    def group_body(gi, carry):
        gb = base_ch + gi * AGC
        pltpu.sync_copy(src_hbm.at[pl.ds(gb, AGC)], sidx_v)
        pltpu.sync_copy(dst_hbm.at[pl.ds(gb, AGC)], didx_v)
        pltpu.async_copy(g_hbm.at[sidx_v.at[0]], rows_a, sem_a)

        def chunk_pair(j2, carry2):
            j = j2 * 2
            pltpu.make_async_copy(g_hbm.at[sidx_v.at[j]], rows_a, sem_a).wait()
            pltpu.async_copy(g_hbm.at[sidx_v.at[j + 1]], rows_b, sem_b)
            pltpu.sync_copy(rows_a, agg_sh.at[didx_v.at[j]], add=True)
            pltpu.make_async_copy(g_hbm.at[sidx_v.at[j + 1]], rows_b, sem_b).wait()

            @pl.when(j + 2 < AGC)
            def _():
                pltpu.async_copy(g_hbm.at[sidx_v.at[j + 2]], rows_a, sem_a)

            pltpu.sync_copy(rows_b, agg_sh.at[didx_v.at[j + 1]], add=True)
            return carry2

        lax.fori_loop(0, AGC // 2, chunk_pair, 0)
        return carry

    lax.fori_loop(0, nch // AGC, group_body, 0)
    plsc.subcore_barrier()

    for k in range(ROWS_PER_TILE // AC):
        pltpu.sync_copy(agg_sh.at[pl.ds(base + k * AC, AC)], rows_a)
        pltpu.sync_copy(rows_a, out_hbm.at[c, pl.ds(base + k * AC, AC)])


@functools.cache
def _agg_call():
    return pl.kernel(
        _agg_body,
        out_type=jax.ShapeDtypeStruct((NC, NP, D), _F32),
        mesh=_mesh(),
        scratch_types=[
            pltpu.VMEM((AGC, AC), jnp.int32),
            pltpu.VMEM((AGC, AC), jnp.int32),
            pltpu.VMEM((AC, D), _F32),
            pltpu.VMEM((AC, D), _F32),
            pltpu.VMEM_SHARED((NP, D), _F32),
            pltpu.SemaphoreType.DMA,
            pltpu.SemaphoreType.DMA,
        ],
    )


_R = 1024


def _tc1_body(x_ref, pw_ref, pb_ref, w1_ref, degp_ref, g_ref, dinv_ref):
    h = jnp.dot(x_ref[...], pw_ref[...], preferred_element_type=_F32)
    h = h + pb_ref[...][None, :]
    deg = 1.0 + jnp.sum(degp_ref[...], axis=0)
    dinv = lax.rsqrt(deg)
    g_ref[...] = jnp.dot(h, w1_ref[...], preferred_element_type=_F32) * dinv[:, None]
    dinv_ref[...] = dinv[:, None]


_tc1_call = pl.pallas_call(
    _tc1_body,
    grid=(NP // _R,),
    in_specs=[
        pl.BlockSpec((_R, D), lambda i: (i, 0)),
        pl.BlockSpec((D, D), lambda i: (0, 0)),
        pl.BlockSpec((D,), lambda i: (0,)),
        pl.BlockSpec((D, D), lambda i: (0, 0)),
        pl.BlockSpec((NW, _R), lambda i: (0, i)),
    ],
    out_specs=[
        pl.BlockSpec((_R, D), lambda i: (i, 0)),
        pl.BlockSpec((_R, 1), lambda i: (i, 0)),
    ],
    out_shape=[
        jax.ShapeDtypeStruct((NP, D), _F32),
        jax.ShapeDtypeStruct((NP, 1), _F32),
    ],
)


def _tc_mid_body(p_ref, g_ref, dinv_ref, b_ref, w_ref, gout_ref):
    p = p_ref[...]
    agg = (p[0] + p[1] + g_ref[...]) * dinv_ref[...]
    h = jnp.maximum(agg + b_ref[...][None, :], 0.0)
    gout_ref[...] = jnp.dot(h, w_ref[...], preferred_element_type=_F32) * dinv_ref[...]


_tc_mid_call = pl.pallas_call(
    _tc_mid_body,
    grid=(NP // _R,),
    in_specs=[
        pl.BlockSpec((NC, _R, D), lambda i: (0, i, 0)),
        pl.BlockSpec((_R, D), lambda i: (i, 0)),
        pl.BlockSpec((_R, 1), lambda i: (i, 0)),
        pl.BlockSpec((D,), lambda i: (0,)),
        pl.BlockSpec((D, D), lambda i: (0, 0)),
    ],
    out_specs=pl.BlockSpec((_R, D), lambda i: (i, 0)),
    out_shape=jax.ShapeDtypeStruct((NP, D), _F32),
)


def _tc_out_body(p_ref, g_ref, dinv_ref, b_ref, w_ref, ob_ref, out_ref):
    p = p_ref[...]
    agg = (p[0] + p[1] + g_ref[...]) * dinv_ref[...]
    h = jnp.maximum(agg + b_ref[...][None, :], 0.0)
    out_ref[...] = (
        jnp.dot(h, w_ref[...], preferred_element_type=_F32) + ob_ref[...][None, :]
    )


_tc_out_call = pl.pallas_call(
    _tc_out_body,
    grid=(NP // _R,),
    in_specs=[
        pl.BlockSpec((NC, _R, D), lambda i: (0, i, 0)),
        pl.BlockSpec((_R, D), lambda i: (i, 0)),
        pl.BlockSpec((_R, 1), lambda i: (i, 0)),
        pl.BlockSpec((D,), lambda i: (0,)),
        pl.BlockSpec((D, D), lambda i: (0, 0)),
        pl.BlockSpec((D,), lambda i: (0,)),
    ],
    out_specs=pl.BlockSpec((_R, D), lambda i: (i, 0)),
    out_shape=jax.ShapeDtypeStruct((NP, D), _F32),
)


def kernel(x, edge_index, proj_W, proj_b, W1, b1, W2, b2, out_W, out_b):
    src = edge_index[0].astype(jnp.int32)
    dst = edge_index[1].astype(jnp.int32)
    pad_e = EP - N_EDGES
    pad_idx = jnp.full((pad_e,), N_NODES, jnp.int32)
    src_flat = jnp.concatenate([src, pad_idx])
    dst_flat = jnp.concatenate([dst, pad_idx])
    dstp_deg = dst_flat.reshape(NW * CPW, CHUNK)
    srcp = src_flat.reshape(NW * CPW, AC)
    dstp = dst_flat.reshape(NW * CPW, AC)
    xp = jnp.pad(x, ((0, NP - N_NODES), (0, 0)))

    degp = _deg_call()(dstp_deg)
    g1, dinv = _tc1_call(xp, proj_W, proj_b, W1, degp)
    parts1 = _agg_call()(g1, srcp, dstp)
    g2 = _tc_mid_call(parts1, g1, dinv, b1, W2)
    parts2 = _agg_call()(g2, srcp, dstp)
    out = _tc_out_call(parts2, g2, dinv, b2, out_W, out_b)
    return out[:N_NODES]
